# baseline (speedup 1.0000x reference)
import functools

import jax
import jax.numpy as jnp
from jax.experimental import pallas as pl
from jax.experimental.pallas import tpu as pltpu
from jax.experimental.pallas import tpu_sc as plsc

B, T, D = 2, 2048, 1024
NR, NS, TOPK = 16, 2, 2
M = B * T
TM1 = 512
TMG = 256
NT = 48
XROWS = NT * TMG
NC, NSC = 2, 16
NW = NC * NSC
CHUNK = M // NW

def _sc_mesh():
    return plsc.VectorSubcoreMesh(
        core_axis_name="c", subcore_axis_name="s",
        num_cores=NC, num_subcores=NSC)


def _k1_body(cent_ref, u_ref, pos0_ref, pos1_ref, v0_ref, v1_ref,
             gmat_ref, gm_ref, cnt_ref, off_ref, run_ref):
    p = pl.program_id(0)
    j = pl.program_id(1)

    u = u_ref[...]
    s = jax.nn.sigmoid(
        jnp.dot(u, cent_ref[...], preferred_element_type=jnp.float32))
    idx = jax.lax.broadcasted_iota(jnp.int32, s.shape, 1)
    m1 = jnp.max(s, axis=1, keepdims=True)
    i1 = jnp.min(jnp.where(s == m1, idx, NR), axis=1, keepdims=True)
    s2 = jnp.where(idx == i1, -jnp.inf, s)
    m2 = jnp.max(s2, axis=1, keepdims=True)
    i2 = jnp.min(jnp.where(s2 == m2, idx, NR), axis=1, keepdims=True)
    a0 = (idx == i1)
    a1 = (idx == i2)
    occ = a0.astype(jnp.float32) + a1.astype(jnp.float32)

    @pl.when(jnp.logical_and(p == 0, j == 0))
    def _():
        cnt_ref[...] = jnp.zeros_like(cnt_ref)

    @pl.when(p == 0)
    def _():
        cnt_ref[...] = cnt_ref[...] + jnp.sum(occ, axis=0, keepdims=True)

    @pl.when(jnp.logical_and(p == 1, j == 0))
    def _():
        cnt = cnt_ref[...].astype(jnp.int32)
        padded = ((cnt + (TMG - 1)) // TMG) * TMG
        r16 = jax.lax.broadcasted_iota(jnp.int32, (NR, NR), 0)
        c16 = jax.lax.broadcasted_iota(jnp.int32, (NR, NR), 1)
        tri = (r16 <= c16).astype(jnp.float32)
        incl = jnp.dot(padded.astype(jnp.float32), tri,
                       preferred_element_type=jnp.float32).astype(jnp.int32)
        off_ref[...] = (incl - padded).astype(jnp.float32)
        run_ref[...] = jnp.zeros_like(run_ref)
        wio = jax.lax.broadcasted_iota(jnp.int32, (NT, NR), 0)
        gm = jnp.sum((incl <= wio * TMG).astype(jnp.int32),
                     axis=1, keepdims=True)
        gm_ref[...] = jnp.minimum(gm, NR - 1)

    @pl.when(p == 1)
    def _():
        rr = jax.lax.broadcasted_iota(jnp.int32, (TM1, TM1), 0)
        cc = jax.lax.broadcasted_iota(jnp.int32, (TM1, TM1), 1)
        tril = (rr >= cc).astype(jnp.float32)
        cum = jnp.dot(tril, occ, preferred_element_type=jnp.float32)
        excl = cum - occ
        base = off_ref[...] + run_ref[...]
        pmat = base + excl
        a0f = a0.astype(jnp.float32)
        a1f = a1.astype(jnp.float32)
        pos0_ref[...] = jnp.sum(a0f * pmat, axis=1,
                                keepdims=True).astype(jnp.int32)
        pos1_ref[...] = jnp.sum(a1f * (pmat + a0f), axis=1,
                                keepdims=True).astype(jnp.int32)
        v0_ref[...] = jnp.sum(jnp.where(a0, s, 0.0), axis=1, keepdims=True)
        v1_ref[...] = jnp.sum(jnp.where(a1, s, 0.0), axis=1, keepdims=True)
        gmat_ref[...] = jnp.where(a0 | a1, s, 0.0)
        run_ref[...] = run_ref[...] + jnp.sum(occ, axis=0, keepdims=True)


def _k1(uf, centT):
    return pl.pallas_call(
        _k1_body,
        grid=(2, M // TM1),
        in_specs=[
            pl.BlockSpec((D, NR), lambda p, j: (0, 0)),
            pl.BlockSpec((TM1, D), lambda p, j: (j, 0)),
        ],
        out_specs=[
            pl.BlockSpec((TM1, 1), lambda p, j: (j, 0)),
            pl.BlockSpec((TM1, 1), lambda p, j: (j, 0)),
            pl.BlockSpec((TM1, 1), lambda p, j: (j, 0)),
            pl.BlockSpec((TM1, 1), lambda p, j: (j, 0)),
            pl.BlockSpec((TM1, NR), lambda p, j: (j, 0)),
            pl.BlockSpec((NT, 1), lambda p, j: (0, 0)),
        ],
        out_shape=[
            jax.ShapeDtypeStruct((M, 1), jnp.int32),
            jax.ShapeDtypeStruct((M, 1), jnp.int32),
            jax.ShapeDtypeStruct((M, 1), jnp.float32),
            jax.ShapeDtypeStruct((M, 1), jnp.float32),
            jax.ShapeDtypeStruct((M, NR), jnp.float32),
            jax.ShapeDtypeStruct((NT, 1), jnp.int32),
        ],
        scratch_shapes=[
            pltpu.VMEM((1, NR), jnp.float32),
            pltpu.VMEM((1, NR), jnp.float32),
            pltpu.VMEM((1, NR), jnp.float32),
        ],
    )(centT, uf)


HCH = CHUNK // 2
QCH = CHUNK // 4


def _sc_scatter(u3, pos0, pos1):
    @functools.partial(
        pl.kernel,
        out_type=jax.ShapeDtypeStruct((XROWS, 8, 128), jnp.float32),
        mesh=_sc_mesh(),
        scratch_types=[
            pltpu.VMEM((HCH,), jnp.int32),
            pltpu.VMEM((HCH,), jnp.int32),
            pltpu.VMEM((HCH, 8, 128), jnp.float32),
            pltpu.SemaphoreType.DMA,
            pltpu.SemaphoreType.DMA,
        ],
    )
    def k(u_hbm, p0_hbm, p1_hbm, x_hbm, idx0_v, idx1_v, rows_v, sem0, sem1):
        wid = jax.lax.axis_index("c") * NSC + jax.lax.axis_index("s")
        for h in range(2):
            base = wid * CHUNK + h * HCH
            pltpu.sync_copy(p0_hbm.at[pl.ds(base, HCH)], idx0_v)
            pltpu.sync_copy(p1_hbm.at[pl.ds(base, HCH)], idx1_v)
            pltpu.sync_copy(u_hbm.at[pl.ds(base, HCH)], rows_v)
            c0 = pltpu.make_async_copy(rows_v, x_hbm.at[idx0_v], sem0)
            c1 = pltpu.make_async_copy(rows_v, x_hbm.at[idx1_v], sem1)
            c0.start()
            c1.start()
            c0.wait()
            c1.wait()

    return k(u3, pos0, pos1)


def _k3_body(gm_ref, x_ref, w_ref, y_ref, wbf_ref):
    w = pl.program_id(0)
    g = gm_ref[w]
    gprev = gm_ref[jnp.maximum(w - 1, 0)]
    changed = jnp.logical_or(w == 0, g != gprev)

    @pl.when(changed)
    def _():
        wbf_ref[...] = w_ref[0].astype(jnp.bfloat16)

    y_ref[...] = jax.lax.dot_general(
        x_ref[...].astype(jnp.bfloat16), wbf_ref[...], (((1,), (1,)), ((), ())),
        preferred_element_type=jnp.float32)


def _k3(gm, x2d, Wr):
    return pl.pallas_call(
        _k3_body,
        grid_spec=pltpu.PrefetchScalarGridSpec(
            num_scalar_prefetch=1,
            grid=(NT,),
            in_specs=[
                pl.BlockSpec((TMG, D), lambda w, gm: (w, 0)),
                pl.BlockSpec((1, D, D), lambda w, gm: (gm[w], 0, 0)),
            ],
            out_specs=pl.BlockSpec((TMG, D), lambda w, gm: (w, 0)),
            scratch_shapes=[pltpu.VMEM((D, D), jnp.bfloat16)],
        ),
        out_shape=jax.ShapeDtypeStruct((XROWS, D), jnp.float32),
    )(gm, x2d, Wr)


def _sc_gather(y3, pos0, pos1):
    @functools.partial(
        pl.kernel,
        out_type=(jax.ShapeDtypeStruct((M, 8, 128), jnp.float32),
                  jax.ShapeDtypeStruct((M, 8, 128), jnp.float32)),
        mesh=_sc_mesh(),
        scratch_types=[
            pltpu.VMEM((QCH,), jnp.int32),
            pltpu.VMEM((QCH,), jnp.int32),
            pltpu.VMEM((QCH, 8, 128), jnp.float32),
            pltpu.VMEM((QCH, 8, 128), jnp.float32),
            pltpu.SemaphoreType.DMA,
            pltpu.SemaphoreType.DMA,
        ],
    )
    def k(y_hbm, p0_hbm, p1_hbm, r0_hbm, r1_hbm,
          idx0_v, idx1_v, rows0_v, rows1_v, sem0, sem1):
        wid = jax.lax.axis_index("c") * NSC + jax.lax.axis_index("s")
        for h in range(CHUNK // QCH):
            base = wid * CHUNK + h * QCH
            pltpu.sync_copy(p0_hbm.at[pl.ds(base, QCH)], idx0_v)
            pltpu.sync_copy(p1_hbm.at[pl.ds(base, QCH)], idx1_v)
            c0 = pltpu.make_async_copy(y_hbm.at[idx0_v], rows0_v, sem0)
            c1 = pltpu.make_async_copy(y_hbm.at[idx1_v], rows1_v, sem1)
            c0.start()
            c1.start()
            c0.wait()
            pltpu.sync_copy(rows0_v, r0_hbm.at[pl.ds(base, QCH)])
            c1.wait()
            pltpu.sync_copy(rows1_v, r1_hbm.at[pl.ds(base, QCH)])

    return k(y3, pos0, pos1)


def _k5_body(br_ref, ws_ref, bsum_ref, u_ref, ubf_ref, gmat_ref,
             v0_ref, v1_ref, r0_ref, r1_ref, o_ref):
    acc = u_ref[...] + bsum_ref[...]
    acc = acc + jnp.dot(gmat_ref[...], br_ref[...],
                        preferred_element_type=jnp.float32)
    acc = acc + jax.lax.dot_general(
        ubf_ref[...], ws_ref[...], (((1,), (1,)), ((), ())),
        preferred_element_type=jnp.float32)
    acc = acc + v0_ref[...] * r0_ref[...]
    acc = acc + v1_ref[...] * r1_ref[...]
    o_ref[...] = acc


def _k5(uf, ubf, gmat, v0, v1, r0, r1, br, wsum_bf, bsum):
    return pl.pallas_call(
        _k5_body,
        grid=(M // TM1,),
        in_specs=[
            pl.BlockSpec((NR, D), lambda j: (0, 0)),
            pl.BlockSpec((D, D), lambda j: (0, 0)),
            pl.BlockSpec((1, D), lambda j: (0, 0)),
            pl.BlockSpec((TM1, D), lambda j: (j, 0)),
            pl.BlockSpec((TM1, D), lambda j: (j, 0)),
            pl.BlockSpec((TM1, NR), lambda j: (j, 0)),
            pl.BlockSpec((TM1, 1), lambda j: (j, 0)),
            pl.BlockSpec((TM1, 1), lambda j: (j, 0)),
            pl.BlockSpec((TM1, D), lambda j: (j, 0)),
            pl.BlockSpec((TM1, D), lambda j: (j, 0)),
        ],
        out_specs=pl.BlockSpec((TM1, D), lambda j: (j, 0)),
        out_shape=jax.ShapeDtypeStruct((M, D), jnp.float32),
    )(br, wsum_bf, bsum, uf, ubf, gmat, v0, v1, r0, r1)


def kernel(u, centroids, Wr, br, Ws, bs):
    uf = u.reshape(M, D)
    ubf = uf.astype(jnp.bfloat16)
    centT = centroids.T
    wsum_bf = (Ws[0] + Ws[1]).astype(jnp.bfloat16)
    bsum = (bs[0] + bs[1]).reshape(1, D)

    pos0, pos1, v0, v1, gmat, gm = _k1(uf, centT)
    pos0f = pos0.reshape(M)
    pos1f = pos1.reshape(M)

    x3 = _sc_scatter(uf.reshape(M, 8, 128), pos0f, pos1f)
    y2d = _k3(gm.reshape(NT), x3.reshape(XROWS, D), Wr)
    r0, r1 = _sc_gather(y2d.reshape(XROWS, 8, 128), pos0f, pos1f)

    out = _k5(uf, ubf, gmat, v0, v1,
              r0.reshape(M, D), r1.reshape(M, D), br, wsum_bf, bsum)
    return out.reshape(B, T, D)


# speedup vs baseline: 1.4905x; 1.4905x over previous
import functools

import jax
import jax.numpy as jnp
from jax.experimental import pallas as pl
from jax.experimental.pallas import tpu as pltpu
from jax.experimental.pallas import tpu_sc as plsc

B, T, D = 2, 2048, 1024
NR, NS, TOPK = 16, 2, 2
M = B * T
TM1 = 512
TMG = 256
NT = 48
XROWS = NT * TMG
NC, NSC = 2, 16
NW = NC * NSC
CHUNK = M // NW

def _sc_mesh():
    return plsc.VectorSubcoreMesh(
        core_axis_name="c", subcore_axis_name="s",
        num_cores=NC, num_subcores=NSC)


def _k1_body(cent_ref, u_ref, pos0_ref, pos1_ref, v0_ref, v1_ref,
             gmat_ref, gm_ref, cnt_ref, off_ref, run_ref):
    p = pl.program_id(0)
    j = pl.program_id(1)

    u = u_ref[...]
    s = jax.nn.sigmoid(
        jnp.dot(u, cent_ref[...], preferred_element_type=jnp.float32))
    idx = jax.lax.broadcasted_iota(jnp.int32, s.shape, 1)
    m1 = jnp.max(s, axis=1, keepdims=True)
    i1 = jnp.min(jnp.where(s == m1, idx, NR), axis=1, keepdims=True)
    s2 = jnp.where(idx == i1, -jnp.inf, s)
    m2 = jnp.max(s2, axis=1, keepdims=True)
    i2 = jnp.min(jnp.where(s2 == m2, idx, NR), axis=1, keepdims=True)
    a0 = (idx == i1)
    a1 = (idx == i2)
    occ = a0.astype(jnp.float32) + a1.astype(jnp.float32)

    @pl.when(jnp.logical_and(p == 0, j == 0))
    def _():
        cnt_ref[...] = jnp.zeros_like(cnt_ref)

    @pl.when(p == 0)
    def _():
        cnt_ref[...] = cnt_ref[...] + jnp.sum(occ, axis=0, keepdims=True)

    @pl.when(jnp.logical_and(p == 1, j == 0))
    def _():
        cnt = cnt_ref[...].astype(jnp.int32)
        padded = ((cnt + (TMG - 1)) // TMG) * TMG
        r16 = jax.lax.broadcasted_iota(jnp.int32, (NR, NR), 0)
        c16 = jax.lax.broadcasted_iota(jnp.int32, (NR, NR), 1)
        tri = (r16 <= c16).astype(jnp.float32)
        incl = jnp.dot(padded.astype(jnp.float32), tri,
                       preferred_element_type=jnp.float32).astype(jnp.int32)
        off_ref[...] = (incl - padded).astype(jnp.float32)
        run_ref[...] = jnp.zeros_like(run_ref)
        wio = jax.lax.broadcasted_iota(jnp.int32, (NT, NR), 0)
        gm = jnp.sum((incl <= wio * TMG).astype(jnp.int32),
                     axis=1, keepdims=True)
        gm_ref[...] = jnp.minimum(gm, NR - 1)

    @pl.when(p == 1)
    def _():
        rr = jax.lax.broadcasted_iota(jnp.int32, (TM1, TM1), 0)
        cc = jax.lax.broadcasted_iota(jnp.int32, (TM1, TM1), 1)
        tril = (rr >= cc).astype(jnp.float32)
        cum = jnp.dot(tril, occ, preferred_element_type=jnp.float32)
        excl = cum - occ
        base = off_ref[...] + run_ref[...]
        pmat = base + excl
        a0f = a0.astype(jnp.float32)
        a1f = a1.astype(jnp.float32)
        pos0_ref[...] = jnp.sum(a0f * pmat, axis=1,
                                keepdims=True).astype(jnp.int32)
        pos1_ref[...] = jnp.sum(a1f * (pmat + a0f), axis=1,
                                keepdims=True).astype(jnp.int32)
        v0_ref[...] = jnp.sum(jnp.where(a0, s, 0.0), axis=1, keepdims=True)
        v1_ref[...] = jnp.sum(jnp.where(a1, s, 0.0), axis=1, keepdims=True)
        gmat_ref[...] = jnp.where(a0 | a1, s, 0.0)
        run_ref[...] = run_ref[...] + jnp.sum(occ, axis=0, keepdims=True)


def _k1(uf, centT):
    return pl.pallas_call(
        _k1_body,
        grid=(2, M // TM1),
        in_specs=[
            pl.BlockSpec((D, NR), lambda p, j: (0, 0)),
            pl.BlockSpec((TM1, D), lambda p, j: (j, 0)),
        ],
        out_specs=[
            pl.BlockSpec((TM1, 1), lambda p, j: (j, 0)),
            pl.BlockSpec((TM1, 1), lambda p, j: (j, 0)),
            pl.BlockSpec((TM1, 1), lambda p, j: (j, 0)),
            pl.BlockSpec((TM1, 1), lambda p, j: (j, 0)),
            pl.BlockSpec((TM1, NR), lambda p, j: (j, 0)),
            pl.BlockSpec((NT, 1), lambda p, j: (0, 0)),
        ],
        out_shape=[
            jax.ShapeDtypeStruct((M, 1), jnp.int32),
            jax.ShapeDtypeStruct((M, 1), jnp.int32),
            jax.ShapeDtypeStruct((M, 1), jnp.float32),
            jax.ShapeDtypeStruct((M, 1), jnp.float32),
            jax.ShapeDtypeStruct((M, NR), jnp.float32),
            jax.ShapeDtypeStruct((NT, 1), jnp.int32),
        ],
        scratch_shapes=[
            pltpu.VMEM((1, NR), jnp.float32),
            pltpu.VMEM((1, NR), jnp.float32),
            pltpu.VMEM((1, NR), jnp.float32),
        ],
    )(centT, uf)


HCH = CHUNK // 2
QCH = CHUNK // 4


def _sc_scatter(u3, pos0, pos1):
    @functools.partial(
        pl.kernel,
        out_type=jax.ShapeDtypeStruct((XROWS, D), jnp.float32),
        mesh=_sc_mesh(),
        scratch_types=[
            pltpu.VMEM((HCH,), jnp.int32),
            pltpu.VMEM((HCH,), jnp.int32),
            pltpu.VMEM((HCH, D), jnp.float32),
            pltpu.SemaphoreType.DMA,
            pltpu.SemaphoreType.DMA,
        ],
    )
    def k(u_hbm, p0_hbm, p1_hbm, x_hbm, idx0_v, idx1_v, rows_v, sem0, sem1):
        wid = jax.lax.axis_index("c") * NSC + jax.lax.axis_index("s")
        for h in range(2):
            base = wid * CHUNK + h * HCH
            pltpu.sync_copy(p0_hbm.at[pl.ds(base, HCH)], idx0_v)
            pltpu.sync_copy(p1_hbm.at[pl.ds(base, HCH)], idx1_v)
            pltpu.sync_copy(u_hbm.at[pl.ds(base, HCH)], rows_v)
            c0 = pltpu.make_async_copy(rows_v, x_hbm.at[idx0_v], sem0)
            c1 = pltpu.make_async_copy(rows_v, x_hbm.at[idx1_v], sem1)
            c0.start()
            c1.start()
            c0.wait()
            c1.wait()

    return k(u3, pos0, pos1)


def _k3_body(gm_ref, x_ref, w_ref, y_ref, wbf_ref):
    w = pl.program_id(0)
    g = gm_ref[w]
    gprev = gm_ref[jnp.maximum(w - 1, 0)]
    changed = jnp.logical_or(w == 0, g != gprev)

    @pl.when(changed)
    def _():
        wbf_ref[...] = w_ref[0].astype(jnp.bfloat16)

    y_ref[...] = jax.lax.dot_general(
        x_ref[...].astype(jnp.bfloat16), wbf_ref[...], (((1,), (1,)), ((), ())),
        preferred_element_type=jnp.float32)


def _k3(gm, x2d, Wr):
    return pl.pallas_call(
        _k3_body,
        grid_spec=pltpu.PrefetchScalarGridSpec(
            num_scalar_prefetch=1,
            grid=(NT,),
            in_specs=[
                pl.BlockSpec((TMG, D), lambda w, gm: (w, 0)),
                pl.BlockSpec((1, D, D), lambda w, gm: (gm[w], 0, 0)),
            ],
            out_specs=pl.BlockSpec((TMG, D), lambda w, gm: (w, 0)),
            scratch_shapes=[pltpu.VMEM((D, D), jnp.bfloat16)],
        ),
        out_shape=jax.ShapeDtypeStruct((XROWS, D), jnp.float32),
    )(gm, x2d, Wr)


def _sc_gather(y3, pos0, pos1):
    @functools.partial(
        pl.kernel,
        out_type=(jax.ShapeDtypeStruct((M, D), jnp.float32),
                  jax.ShapeDtypeStruct((M, D), jnp.float32)),
        mesh=_sc_mesh(),
        scratch_types=[
            pltpu.VMEM((QCH,), jnp.int32),
            pltpu.VMEM((QCH,), jnp.int32),
            pltpu.VMEM((QCH, D), jnp.float32),
            pltpu.VMEM((QCH, D), jnp.float32),
            pltpu.SemaphoreType.DMA,
            pltpu.SemaphoreType.DMA,
        ],
    )
    def k(y_hbm, p0_hbm, p1_hbm, r0_hbm, r1_hbm,
          idx0_v, idx1_v, rows0_v, rows1_v, sem0, sem1):
        wid = jax.lax.axis_index("c") * NSC + jax.lax.axis_index("s")
        for h in range(CHUNK // QCH):
            base = wid * CHUNK + h * QCH
            pltpu.sync_copy(p0_hbm.at[pl.ds(base, QCH)], idx0_v)
            pltpu.sync_copy(p1_hbm.at[pl.ds(base, QCH)], idx1_v)
            c0 = pltpu.make_async_copy(y_hbm.at[idx0_v], rows0_v, sem0)
            c1 = pltpu.make_async_copy(y_hbm.at[idx1_v], rows1_v, sem1)
            c0.start()
            c1.start()
            c0.wait()
            pltpu.sync_copy(rows0_v, r0_hbm.at[pl.ds(base, QCH)])
            c1.wait()
            pltpu.sync_copy(rows1_v, r1_hbm.at[pl.ds(base, QCH)])

    return k(y3, pos0, pos1)


def _k5_body(br_ref, ws_ref, bsum_ref, u_ref, ubf_ref, gmat_ref,
             v0_ref, v1_ref, r0_ref, r1_ref, o_ref):
    acc = u_ref[...] + bsum_ref[...]
    acc = acc + jnp.dot(gmat_ref[...], br_ref[...],
                        preferred_element_type=jnp.float32)
    acc = acc + jax.lax.dot_general(
        ubf_ref[...], ws_ref[...], (((1,), (1,)), ((), ())),
        preferred_element_type=jnp.float32)
    acc = acc + v0_ref[...] * r0_ref[...]
    acc = acc + v1_ref[...] * r1_ref[...]
    o_ref[...] = acc


def _k5(uf, ubf, gmat, v0, v1, r0, r1, br, wsum_bf, bsum):
    return pl.pallas_call(
        _k5_body,
        grid=(M // TM1,),
        in_specs=[
            pl.BlockSpec((NR, D), lambda j: (0, 0)),
            pl.BlockSpec((D, D), lambda j: (0, 0)),
            pl.BlockSpec((1, D), lambda j: (0, 0)),
            pl.BlockSpec((TM1, D), lambda j: (j, 0)),
            pl.BlockSpec((TM1, D), lambda j: (j, 0)),
            pl.BlockSpec((TM1, NR), lambda j: (j, 0)),
            pl.BlockSpec((TM1, 1), lambda j: (j, 0)),
            pl.BlockSpec((TM1, 1), lambda j: (j, 0)),
            pl.BlockSpec((TM1, D), lambda j: (j, 0)),
            pl.BlockSpec((TM1, D), lambda j: (j, 0)),
        ],
        out_specs=pl.BlockSpec((TM1, D), lambda j: (j, 0)),
        out_shape=jax.ShapeDtypeStruct((M, D), jnp.float32),
    )(br, wsum_bf, bsum, uf, ubf, gmat, v0, v1, r0, r1)


def kernel(u, centroids, Wr, br, Ws, bs):
    uf = u.reshape(M, D)
    ubf = uf.astype(jnp.bfloat16)
    centT = centroids.T
    wsum_bf = (Ws[0] + Ws[1]).astype(jnp.bfloat16)
    bsum = (bs[0] + bs[1]).reshape(1, D)

    pos0, pos1, v0, v1, gmat, gm = _k1(uf, centT)
    pos0f = pos0.reshape(M)
    pos1f = pos1.reshape(M)

    x2d = _sc_scatter(uf, pos0f, pos1f)
    y2d = _k3(gm.reshape(NT), x2d, Wr)
    r0, r1 = _sc_gather(y2d, pos0f, pos1f)

    out = _k5(uf, ubf, gmat, v0, v1, r0, r1, br, wsum_bf, bsum)
    return out.reshape(B, T, D)


# speedup vs baseline: 1.5209x; 1.0204x over previous
import functools

import jax
import jax.numpy as jnp
from jax.experimental import pallas as pl
from jax.experimental.pallas import tpu as pltpu
from jax.experimental.pallas import tpu_sc as plsc

B, T, D = 2, 2048, 1024
NR, NS, TOPK = 16, 2, 2
M = B * T
TM1 = 512
NJ = M // TM1
TMG = 256
NT = 48
NTP = 64
XROWS = NT * TMG
NC, NSC = 2, 16
NW = NC * NSC
CHUNK = M // NW
HCH = CHUNK // 2
QCH = CHUNK // 4


def _sc_mesh():
    return plsc.VectorSubcoreMesh(
        core_axis_name="c", subcore_axis_name="s",
        num_cores=NC, num_subcores=NSC)


def _k1a_body(cent_ref, ws_ref, br_ref, bsum_ref, u_ref,
              base_ref, gmat_ref, cnt_ref):
    u = u_ref[...]
    s = jax.nn.sigmoid(
        jnp.dot(u, cent_ref[...], preferred_element_type=jnp.float32))
    idx = jax.lax.broadcasted_iota(jnp.int32, s.shape, 1)
    m1 = jnp.max(s, axis=1, keepdims=True)
    i1 = jnp.min(jnp.where(s == m1, idx, NR), axis=1, keepdims=True)
    s2 = jnp.where(idx == i1, -jnp.inf, s)
    m2 = jnp.max(s2, axis=1, keepdims=True)
    i2 = jnp.min(jnp.where(s2 == m2, idx, NR), axis=1, keepdims=True)
    keep = (idx == i1) | (idx == i2)
    gmat = jnp.where(keep, s, 0.0)
    gmat_ref[...] = gmat
    cnt_ref[0] = jnp.sum(keep.astype(jnp.float32), axis=0, keepdims=True)

    acc = u + bsum_ref[...] + jnp.dot(gmat, br_ref[...],
                                      preferred_element_type=jnp.float32)
    acc = acc + jax.lax.dot_general(
        u.astype(jnp.bfloat16), ws_ref[...], (((1,), (1,)), ((), ())),
        preferred_element_type=jnp.float32)
    base_ref[...] = acc


def _k1a(uf, centT, wsum_bf, br, bsum):
    return pl.pallas_call(
        _k1a_body,
        grid=(NJ,),
        in_specs=[
            pl.BlockSpec((D, NR), lambda j: (0, 0)),
            pl.BlockSpec((D, D), lambda j: (0, 0)),
            pl.BlockSpec((NR, D), lambda j: (0, 0)),
            pl.BlockSpec((1, D), lambda j: (0, 0)),
            pl.BlockSpec((TM1, D), lambda j: (j, 0)),
        ],
        out_specs=[
            pl.BlockSpec((TM1, D), lambda j: (j, 0)),
            pl.BlockSpec((TM1, NR), lambda j: (j, 0)),
            pl.BlockSpec((1, 1, NR), lambda j: (j, 0, 0)),
        ],
        out_shape=[
            jax.ShapeDtypeStruct((M, D), jnp.float32),
            jax.ShapeDtypeStruct((M, NR), jnp.float32),
            jax.ShapeDtypeStruct((NJ, 1, NR), jnp.float32),
        ],
    )(centT, wsum_bf, br, bsum, uf)


def _k1b_body(cnt_ref, gmat_ref, pos0_ref, pos1_ref, v0_ref, v1_ref,
              gm_ref, off_ref, run_ref):
    j = pl.program_id(0)

    @pl.when(j == 0)
    def _():
        cnt = jnp.sum(cnt_ref[...], axis=0)
        padded = ((cnt.astype(jnp.int32) + (TMG - 1)) // TMG) * TMG
        r16 = jax.lax.broadcasted_iota(jnp.int32, (NR, NR), 0)
        c16 = jax.lax.broadcasted_iota(jnp.int32, (NR, NR), 1)
        tri = (r16 <= c16).astype(jnp.float32)
        incl = jnp.dot(padded.astype(jnp.float32), tri,
                       preferred_element_type=jnp.float32).astype(jnp.int32)
        off_ref[...] = (incl - padded).astype(jnp.float32)
        run_ref[...] = jnp.zeros_like(run_ref)
        wio = jax.lax.broadcasted_iota(jnp.int32, (NTP, NR), 0)
        gm = jnp.sum((incl <= wio * TMG).astype(jnp.int32),
                     axis=1, keepdims=True)
        used = incl[0, NR - 1] // TMG
        gm = jnp.minimum(gm, NR - 1)
        gmio = jax.lax.broadcasted_iota(jnp.int32, (NTP, 1), 0)
        gm_ref[...] = jnp.where(gmio == NT, used, gm)

    gmat = gmat_ref[...]
    a = (gmat > 0.0).astype(jnp.float32)
    r16 = jax.lax.broadcasted_iota(jnp.int32, (NR, NR), 0)
    c16 = jax.lax.broadcasted_iota(jnp.int32, (NR, NR), 1)
    tri16 = (r16 <= c16).astype(jnp.float32)
    lcum = jnp.dot(a, tri16, preferred_element_type=jnp.float32)
    a0 = a * (lcum == 1.0)
    a1 = a * (lcum == 2.0)
    rr = jax.lax.broadcasted_iota(jnp.int32, (TM1, TM1), 0)
    cc = jax.lax.broadcasted_iota(jnp.int32, (TM1, TM1), 1)
    tril = (rr >= cc).astype(jnp.float32)
    cum = jnp.dot(tril, a, preferred_element_type=jnp.float32)
    excl = cum - a
    pmat = off_ref[...] + run_ref[...] + excl
    pos0_ref[...] = jnp.sum(a0 * pmat, axis=1, keepdims=True).astype(jnp.int32)
    pos1_ref[...] = jnp.sum(a1 * (pmat + a0), axis=1,
                            keepdims=True).astype(jnp.int32)
    v0_ref[...] = jnp.sum(a0 * gmat, axis=1, keepdims=True)
    v1_ref[...] = jnp.sum(a1 * gmat, axis=1, keepdims=True)
    run_ref[...] = run_ref[...] + jnp.sum(a, axis=0, keepdims=True)


def _k1b(cnt3, gmat):
    return pl.pallas_call(
        _k1b_body,
        grid=(NJ,),
        in_specs=[
            pl.BlockSpec((NJ, 1, NR), lambda j: (0, 0, 0)),
            pl.BlockSpec((TM1, NR), lambda j: (j, 0)),
        ],
        out_specs=[
            pl.BlockSpec((TM1, 1), lambda j: (j, 0)),
            pl.BlockSpec((TM1, 1), lambda j: (j, 0)),
            pl.BlockSpec((TM1, 1), lambda j: (j, 0)),
            pl.BlockSpec((TM1, 1), lambda j: (j, 0)),
            pl.BlockSpec((NTP, 1), lambda j: (0, 0)),
        ],
        out_shape=[
            jax.ShapeDtypeStruct((M, 1), jnp.int32),
            jax.ShapeDtypeStruct((M, 1), jnp.int32),
            jax.ShapeDtypeStruct((M, 1), jnp.float32),
            jax.ShapeDtypeStruct((M, 1), jnp.float32),
            jax.ShapeDtypeStruct((NTP, 1), jnp.int32),
        ],
        scratch_shapes=[
            pltpu.VMEM((1, NR), jnp.float32),
            pltpu.VMEM((1, NR), jnp.float32),
        ],
    )(cnt3, gmat)


def _sc_scatter(u2, pos0, pos1):
    @functools.partial(
        pl.kernel,
        out_type=jax.ShapeDtypeStruct((XROWS, D), jnp.float32),
        mesh=_sc_mesh(),
        scratch_types=[
            pltpu.VMEM((HCH,), jnp.int32),
            pltpu.VMEM((HCH,), jnp.int32),
            pltpu.VMEM((HCH, D), jnp.float32),
            pltpu.SemaphoreType.DMA,
            pltpu.SemaphoreType.DMA,
        ],
    )
    def k(u_hbm, p0_hbm, p1_hbm, x_hbm, idx0_v, idx1_v, rows_v, sem0, sem1):
        wid = jax.lax.axis_index("c") * NSC + jax.lax.axis_index("s")
        for h in range(2):
            base = wid * CHUNK + h * HCH
            pltpu.sync_copy(p0_hbm.at[pl.ds(base, HCH)], idx0_v)
            pltpu.sync_copy(p1_hbm.at[pl.ds(base, HCH)], idx1_v)
            pltpu.sync_copy(u_hbm.at[pl.ds(base, HCH)], rows_v)
            c0 = pltpu.make_async_copy(rows_v, x_hbm.at[idx0_v], sem0)
            c1 = pltpu.make_async_copy(rows_v, x_hbm.at[idx1_v], sem1)
            c0.start()
            c1.start()
            c0.wait()
            c1.wait()

    return k(u2, pos0, pos1)


def _k3_body(gm_ref, x_ref, w_ref, y_ref, wbf_ref):
    w = pl.program_id(0)
    g = gm_ref[w]
    gprev = gm_ref[jnp.maximum(w - 1, 0)]
    changed = jnp.logical_or(w == 0, g != gprev)
    valid = w < gm_ref[NT]

    @pl.when(jnp.logical_and(changed, valid))
    def _():
        wbf_ref[...] = w_ref[0].astype(jnp.bfloat16)

    @pl.when(valid)
    def _():
        y_ref[...] = jax.lax.dot_general(
            x_ref[...].astype(jnp.bfloat16), wbf_ref[...],
            (((1,), (1,)), ((), ())), preferred_element_type=jnp.float32)


def _k3(gm, x2d, Wr):
    return pl.pallas_call(
        _k3_body,
        grid_spec=pltpu.PrefetchScalarGridSpec(
            num_scalar_prefetch=1,
            grid=(NT,),
            in_specs=[
                pl.BlockSpec((TMG, D), lambda w, gm: (w, 0)),
                pl.BlockSpec((1, D, D), lambda w, gm: (gm[w], 0, 0)),
            ],
            out_specs=pl.BlockSpec((TMG, D), lambda w, gm: (w, 0)),
            scratch_shapes=[pltpu.VMEM((D, D), jnp.bfloat16)],
        ),
        out_shape=jax.ShapeDtypeStruct((XROWS, D), jnp.float32),
    )(gm, x2d, Wr)


def _sc_gather(y2, pos0, pos1):
    @functools.partial(
        pl.kernel,
        out_type=(jax.ShapeDtypeStruct((M, D), jnp.float32),
                  jax.ShapeDtypeStruct((M, D), jnp.float32)),
        mesh=_sc_mesh(),
        scratch_types=[
            pltpu.VMEM((QCH,), jnp.int32),
            pltpu.VMEM((QCH,), jnp.int32),
            pltpu.VMEM((QCH, D), jnp.float32),
            pltpu.VMEM((QCH, D), jnp.float32),
            pltpu.SemaphoreType.DMA,
            pltpu.SemaphoreType.DMA,
        ],
    )
    def k(y_hbm, p0_hbm, p1_hbm, r0_hbm, r1_hbm,
          idx0_v, idx1_v, rows0_v, rows1_v, sem0, sem1):
        wid = jax.lax.axis_index("c") * NSC + jax.lax.axis_index("s")
        for h in range(CHUNK // QCH):
            base = wid * CHUNK + h * QCH
            pltpu.sync_copy(p0_hbm.at[pl.ds(base, QCH)], idx0_v)
            pltpu.sync_copy(p1_hbm.at[pl.ds(base, QCH)], idx1_v)
            c0 = pltpu.make_async_copy(y_hbm.at[idx0_v], rows0_v, sem0)
            c1 = pltpu.make_async_copy(y_hbm.at[idx1_v], rows1_v, sem1)
            c0.start()
            c1.start()
            c0.wait()
            pltpu.sync_copy(rows0_v, r0_hbm.at[pl.ds(base, QCH)])
            c1.wait()
            pltpu.sync_copy(rows1_v, r1_hbm.at[pl.ds(base, QCH)])

    return k(y2, pos0, pos1)


def _k5_body(base_ref, v0_ref, v1_ref, r0_ref, r1_ref, o_ref):
    o_ref[...] = (base_ref[...] + v0_ref[...] * r0_ref[...]
                  + v1_ref[...] * r1_ref[...])


def _k5(base, v0, v1, r0, r1):
    return pl.pallas_call(
        _k5_body,
        grid=(NJ,),
        in_specs=[
            pl.BlockSpec((TM1, D), lambda j: (j, 0)),
            pl.BlockSpec((TM1, 1), lambda j: (j, 0)),
            pl.BlockSpec((TM1, 1), lambda j: (j, 0)),
            pl.BlockSpec((TM1, D), lambda j: (j, 0)),
            pl.BlockSpec((TM1, D), lambda j: (j, 0)),
        ],
        out_specs=pl.BlockSpec((TM1, D), lambda j: (j, 0)),
        out_shape=jax.ShapeDtypeStruct((M, D), jnp.float32),
    )(base, v0, v1, r0, r1)


def kernel(u, centroids, Wr, br, Ws, bs):
    uf = u.reshape(M, D)
    centT = centroids.T
    wsum_bf = (Ws[0] + Ws[1]).astype(jnp.bfloat16)
    bsum = (bs[0] + bs[1]).reshape(1, D)

    base, gmat, cnt3 = _k1a(uf, centT, wsum_bf, br, bsum)
    pos0, pos1, v0, v1, gm = _k1b(cnt3, gmat)
    pos0f = pos0.reshape(M)
    pos1f = pos1.reshape(M)

    x2d = _sc_scatter(uf, pos0f, pos1f)
    y2d = _k3(gm.reshape(NTP), x2d, Wr)
    r0, r1 = _sc_gather(y2d, pos0f, pos1f)

    out = _k5(base, v0, v1, r0, r1)
    return out.reshape(B, T, D)


# speedup vs baseline: 1.8825x; 1.2377x over previous
import functools

import jax
import jax.numpy as jnp
from jax.experimental import pallas as pl
from jax.experimental.pallas import tpu as pltpu
from jax.experimental.pallas import tpu_sc as plsc

B, T, D = 2, 2048, 1024
NR, NS, TOPK = 16, 2, 2
DH = D // 2
M = B * T
TM1 = 512
NJ = M // TM1
TMG = 256
NT = 48
NTP = 64
XROWS = NT * TMG
NC, NSC = 2, 16
NW = NC * NSC
CHUNK = M // NW
HCH = CHUNK // 2


def _sc_mesh():
    return plsc.VectorSubcoreMesh(
        core_axis_name="c", subcore_axis_name="s",
        num_cores=NC, num_subcores=NSC)


def _pack(bf):
    lo = jax.lax.bitcast_convert_type(bf[:, :DH], jnp.uint16).astype(jnp.uint32)
    hi = jax.lax.bitcast_convert_type(bf[:, DH:], jnp.uint16).astype(jnp.uint32)
    return jax.lax.bitcast_convert_type((hi << 16) | lo, jnp.int32)


def _unpack(pk):
    w = jax.lax.bitcast_convert_type(pk, jnp.uint32)
    lo = jax.lax.bitcast_convert_type((w & 0xFFFF).astype(jnp.uint16),
                                      jnp.bfloat16)
    hi = jax.lax.bitcast_convert_type((w >> 16).astype(jnp.uint16),
                                      jnp.bfloat16)
    return lo, hi


def _k1a_body(cent_ref, ws_ref, br_ref, bs_ref, u_ref,
              base_ref, gmat_ref, cnt_ref, upk_ref, wsbf_ref):
    j = pl.program_id(0)

    @pl.when(j == 0)
    def _():
        wsbf_ref[...] = (ws_ref[0] + ws_ref[1]).astype(jnp.bfloat16)

    u = u_ref[...]
    s = jax.nn.sigmoid(
        jnp.dot(u, cent_ref[...], preferred_element_type=jnp.float32))
    idx = jax.lax.broadcasted_iota(jnp.int32, s.shape, 1)
    m1 = jnp.max(s, axis=1, keepdims=True)
    i1 = jnp.min(jnp.where(s == m1, idx, NR), axis=1, keepdims=True)
    s2 = jnp.where(idx == i1, -jnp.inf, s)
    m2 = jnp.max(s2, axis=1, keepdims=True)
    i2 = jnp.min(jnp.where(s2 == m2, idx, NR), axis=1, keepdims=True)
    keep = (idx == i1) | (idx == i2)
    gmat = jnp.where(keep, s, 0.0)
    gmat_ref[...] = gmat
    cnt_ref[0] = jnp.sum(keep.astype(jnp.float32), axis=0, keepdims=True)

    ubf = u.astype(jnp.bfloat16)
    upk_ref[...] = _pack(ubf)

    bsum = bs_ref[0:1, :] + bs_ref[1:2, :]
    acc = u + bsum + jnp.dot(gmat, br_ref[...],
                             preferred_element_type=jnp.float32)
    acc = acc + jax.lax.dot_general(
        ubf, wsbf_ref[...], (((1,), (1,)), ((), ())),
        preferred_element_type=jnp.float32)
    base_ref[...] = acc


def _k1a(uf, centT, Ws, br, bs):
    return pl.pallas_call(
        _k1a_body,
        grid=(NJ,),
        in_specs=[
            pl.BlockSpec((D, NR), lambda j: (0, 0)),
            pl.BlockSpec((NS, D, D), lambda j: (0, 0, 0)),
            pl.BlockSpec((NR, D), lambda j: (0, 0)),
            pl.BlockSpec((NS, D), lambda j: (0, 0)),
            pl.BlockSpec((TM1, D), lambda j: (j, 0)),
        ],
        out_specs=[
            pl.BlockSpec((TM1, D), lambda j: (j, 0)),
            pl.BlockSpec((TM1, NR), lambda j: (j, 0)),
            pl.BlockSpec((1, 1, NR), lambda j: (j, 0, 0)),
            pl.BlockSpec((TM1, DH), lambda j: (j, 0)),
        ],
        out_shape=[
            jax.ShapeDtypeStruct((M, D), jnp.float32),
            jax.ShapeDtypeStruct((M, NR), jnp.float32),
            jax.ShapeDtypeStruct((NJ, 1, NR), jnp.float32),
            jax.ShapeDtypeStruct((M, DH), jnp.int32),
        ],
        scratch_shapes=[pltpu.VMEM((D, D), jnp.bfloat16)],
    )(centT, Ws, br, bs, uf)


def _k1b_body(cnt_ref, gmat_ref, pos0_ref, pos1_ref, v0_ref, v1_ref,
              gm_ref, off_ref, run_ref):
    j = pl.program_id(0)

    @pl.when(j == 0)
    def _():
        cnt = jnp.sum(cnt_ref[...], axis=0)
        padded = ((cnt.astype(jnp.int32) + (TMG - 1)) // TMG) * TMG
        r16 = jax.lax.broadcasted_iota(jnp.int32, (NR, NR), 0)
        c16 = jax.lax.broadcasted_iota(jnp.int32, (NR, NR), 1)
        tri = (r16 <= c16).astype(jnp.float32)
        incl = jnp.dot(padded.astype(jnp.float32), tri,
                       preferred_element_type=jnp.float32).astype(jnp.int32)
        off_ref[...] = (incl - padded).astype(jnp.float32)
        run_ref[...] = jnp.zeros_like(run_ref)
        wio = jax.lax.broadcasted_iota(jnp.int32, (NTP, NR), 0)
        gm = jnp.sum((incl <= wio * TMG).astype(jnp.int32),
                     axis=1, keepdims=True)
        used = incl[0, NR - 1] // TMG
        gm = jnp.minimum(gm, NR - 1)
        gmio = jax.lax.broadcasted_iota(jnp.int32, (NTP, 1), 0)
        gm_ref[...] = jnp.where(gmio == NT, used, gm)

    gmat = gmat_ref[...]
    a = (gmat > 0.0).astype(jnp.float32)
    r16 = jax.lax.broadcasted_iota(jnp.int32, (NR, NR), 0)
    c16 = jax.lax.broadcasted_iota(jnp.int32, (NR, NR), 1)
    tri16 = (r16 <= c16).astype(jnp.float32)
    lcum = jnp.dot(a, tri16, preferred_element_type=jnp.float32)
    a0 = a * (lcum == 1.0)
    a1 = a * (lcum == 2.0)
    rr = jax.lax.broadcasted_iota(jnp.int32, (TM1, TM1), 0)
    cc = jax.lax.broadcasted_iota(jnp.int32, (TM1, TM1), 1)
    tril = (rr >= cc).astype(jnp.float32)
    cum = jnp.dot(tril, a, preferred_element_type=jnp.float32)
    excl = cum - a
    pmat = off_ref[...] + run_ref[...] + excl
    pos0_ref[...] = jnp.sum(a0 * pmat, axis=1, keepdims=True).astype(jnp.int32)
    pos1_ref[...] = jnp.sum(a1 * (pmat + a0), axis=1,
                            keepdims=True).astype(jnp.int32)
    v0_ref[...] = jnp.sum(a0 * gmat, axis=1, keepdims=True)
    v1_ref[...] = jnp.sum(a1 * gmat, axis=1, keepdims=True)
    run_ref[...] = run_ref[...] + jnp.sum(a, axis=0, keepdims=True)


def _k1b(cnt3, gmat):
    return pl.pallas_call(
        _k1b_body,
        grid=(NJ,),
        in_specs=[
            pl.BlockSpec((NJ, 1, NR), lambda j: (0, 0, 0)),
            pl.BlockSpec((TM1, NR), lambda j: (j, 0)),
        ],
        out_specs=[
            pl.BlockSpec((TM1, 1), lambda j: (j, 0)),
            pl.BlockSpec((TM1, 1), lambda j: (j, 0)),
            pl.BlockSpec((TM1, 1), lambda j: (j, 0)),
            pl.BlockSpec((TM1, 1), lambda j: (j, 0)),
            pl.BlockSpec((NTP, 1), lambda j: (0, 0)),
        ],
        out_shape=[
            jax.ShapeDtypeStruct((M, 1), jnp.int32),
            jax.ShapeDtypeStruct((M, 1), jnp.int32),
            jax.ShapeDtypeStruct((M, 1), jnp.float32),
            jax.ShapeDtypeStruct((M, 1), jnp.float32),
            jax.ShapeDtypeStruct((NTP, 1), jnp.int32),
        ],
        scratch_shapes=[
            pltpu.VMEM((1, NR), jnp.float32),
            pltpu.VMEM((1, NR), jnp.float32),
        ],
    )(cnt3, gmat)


def _sc_scatter(upk, pos0, pos1):
    @functools.partial(
        pl.kernel,
        out_type=jax.ShapeDtypeStruct((XROWS, DH), jnp.int32),
        mesh=_sc_mesh(),
        scratch_types=[
            pltpu.VMEM((CHUNK,), jnp.int32),
            pltpu.VMEM((CHUNK,), jnp.int32),
            pltpu.VMEM((CHUNK, DH), jnp.int32),
            pltpu.SemaphoreType.DMA,
            pltpu.SemaphoreType.DMA,
        ],
    )
    def k(u_hbm, p0_hbm, p1_hbm, x_hbm, idx0_v, idx1_v, rows_v, sem0, sem1):
        wid = jax.lax.axis_index("c") * NSC + jax.lax.axis_index("s")
        base = wid * CHUNK
        pltpu.sync_copy(p0_hbm.at[pl.ds(base, CHUNK)], idx0_v)
        pltpu.sync_copy(p1_hbm.at[pl.ds(base, CHUNK)], idx1_v)
        pltpu.sync_copy(u_hbm.at[pl.ds(base, CHUNK)], rows_v)
        c0 = pltpu.make_async_copy(rows_v, x_hbm.at[idx0_v], sem0)
        c1 = pltpu.make_async_copy(rows_v, x_hbm.at[idx1_v], sem1)
        c0.start()
        c1.start()
        c0.wait()
        c1.wait()

    return k(upk, pos0, pos1)


def _k3_body(gm_ref, x_ref, w_ref, y_ref, wbf_ref):
    w = pl.program_id(0)
    g = gm_ref[w]
    gprev = gm_ref[jnp.maximum(w - 1, 0)]
    changed = jnp.logical_or(w == 0, g != gprev)
    valid = w < gm_ref[NT]

    @pl.when(jnp.logical_and(changed, valid))
    def _():
        wbf_ref[...] = w_ref[0].astype(jnp.bfloat16)

    @pl.when(valid)
    def _():
        xlo, xhi = _unpack(x_ref[...])
        wbf = wbf_ref[...]
        acc = jax.lax.dot_general(
            xlo, wbf[:, :DH], (((1,), (1,)), ((), ())),
            preferred_element_type=jnp.float32)
        acc = acc + jax.lax.dot_general(
            xhi, wbf[:, DH:], (((1,), (1,)), ((), ())),
            preferred_element_type=jnp.float32)
        y_ref[...] = _pack(acc.astype(jnp.bfloat16))


def _k3(gm, x2d, Wr):
    return pl.pallas_call(
        _k3_body,
        grid_spec=pltpu.PrefetchScalarGridSpec(
            num_scalar_prefetch=1,
            grid=(NT,),
            in_specs=[
                pl.BlockSpec((TMG, DH), lambda w, gm: (w, 0)),
                pl.BlockSpec((1, D, D), lambda w, gm: (gm[w], 0, 0)),
            ],
            out_specs=pl.BlockSpec((TMG, DH), lambda w, gm: (w, 0)),
            scratch_shapes=[pltpu.VMEM((D, D), jnp.bfloat16)],
        ),
        out_shape=jax.ShapeDtypeStruct((XROWS, DH), jnp.int32),
    )(gm, x2d, Wr)


def _sc_gather(y2, pos0, pos1):
    @functools.partial(
        pl.kernel,
        out_type=(jax.ShapeDtypeStruct((M, DH), jnp.int32),
                  jax.ShapeDtypeStruct((M, DH), jnp.int32)),
        mesh=_sc_mesh(),
        scratch_types=[
            pltpu.VMEM((HCH,), jnp.int32),
            pltpu.VMEM((HCH,), jnp.int32),
            pltpu.VMEM((HCH, DH), jnp.int32),
            pltpu.VMEM((HCH, DH), jnp.int32),
            pltpu.SemaphoreType.DMA,
            pltpu.SemaphoreType.DMA,
        ],
    )
    def k(y_hbm, p0_hbm, p1_hbm, r0_hbm, r1_hbm,
          idx0_v, idx1_v, rows0_v, rows1_v, sem0, sem1):
        wid = jax.lax.axis_index("c") * NSC + jax.lax.axis_index("s")
        for h in range(CHUNK // HCH):
            base = wid * CHUNK + h * HCH
            pltpu.sync_copy(p0_hbm.at[pl.ds(base, HCH)], idx0_v)
            pltpu.sync_copy(p1_hbm.at[pl.ds(base, HCH)], idx1_v)
            c0 = pltpu.make_async_copy(y_hbm.at[idx0_v], rows0_v, sem0)
            c1 = pltpu.make_async_copy(y_hbm.at[idx1_v], rows1_v, sem1)
            c0.start()
            c1.start()
            c0.wait()
            pltpu.sync_copy(rows0_v, r0_hbm.at[pl.ds(base, HCH)])
            c1.wait()
            pltpu.sync_copy(rows1_v, r1_hbm.at[pl.ds(base, HCH)])

    return k(y2, pos0, pos1)


def _k5_body(base_ref, v0_ref, v1_ref, r0_ref, r1_ref, o_ref):
    r0lo, r0hi = _unpack(r0_ref[...])
    r1lo, r1hi = _unpack(r1_ref[...])
    v0 = v0_ref[...]
    v1 = v1_ref[...]
    o_ref[:, :DH] = (base_ref[:, :DH] + v0 * r0lo.astype(jnp.float32)
                     + v1 * r1lo.astype(jnp.float32))
    o_ref[:, DH:] = (base_ref[:, DH:] + v0 * r0hi.astype(jnp.float32)
                     + v1 * r1hi.astype(jnp.float32))


def _k5(base, v0, v1, r0, r1):
    return pl.pallas_call(
        _k5_body,
        grid=(NJ,),
        in_specs=[
            pl.BlockSpec((TM1, D), lambda j: (j, 0)),
            pl.BlockSpec((TM1, 1), lambda j: (j, 0)),
            pl.BlockSpec((TM1, 1), lambda j: (j, 0)),
            pl.BlockSpec((TM1, DH), lambda j: (j, 0)),
            pl.BlockSpec((TM1, DH), lambda j: (j, 0)),
        ],
        out_specs=pl.BlockSpec((TM1, D), lambda j: (j, 0)),
        out_shape=jax.ShapeDtypeStruct((M, D), jnp.float32),
    )(base, v0, v1, r0, r1)


def kernel(u, centroids, Wr, br, Ws, bs):
    uf = u.reshape(M, D)
    centT = centroids.T

    base, gmat, cnt3, upk = _k1a(uf, centT, Ws, br, bs)
    pos0, pos1, v0, v1, gm = _k1b(cnt3, gmat)
    pos0f = pos0.reshape(M)
    pos1f = pos1.reshape(M)

    x2d = _sc_scatter(upk, pos0f, pos1f)
    y2d = _k3(gm.reshape(NTP), x2d, Wr)
    r0, r1 = _sc_gather(y2d, pos0f, pos1f)

    out = _k5(base, v0, v1, r0, r1)
    return out.reshape(B, T, D)
